# Initial kernel scaffold; baseline (speedup 1.0000x reference)
import jax
import jax.numpy as jnp
from jax.experimental import pallas as pl


def kernel(x, token_table, pos_table):
    raise NotImplementedError("write your pallas kernel here")


# speedup vs baseline: 1.0555x; 1.0555x over previous
import functools

import jax
import jax.numpy as jnp
from jax import lax
from jax.experimental import pallas as pl
from jax.experimental.pallas import tpu as pltpu
from jax.experimental.pallas import tpu_sc as plsc

MAXLEN = 200
EMBED_DIM = 32
BATCH = 4096

NC, NS = 2, 16
NW = NC * NS
TOTAL_ROWS = BATCH * MAXLEN
ROWS_PER_W = TOTAL_ROWS // NW
CHUNK = 128
CHUNKS_PER_W = ROWS_PER_W // CHUNK
POS_PAD = MAXLEN + CHUNK


@functools.partial(
    pl.kernel,
    out_type=jax.ShapeDtypeStruct((TOTAL_ROWS, EMBED_DIM), jnp.float32),
    mesh=plsc.VectorSubcoreMesh(core_axis_name="c", subcore_axis_name="s"),
    compiler_params=pltpu.CompilerParams(use_tc_tiling_on_sc=False),
    scratch_types=[
        pltpu.VMEM((CHUNKS_PER_W, CHUNK), jnp.int32),
        pltpu.VMEM((POS_PAD, EMBED_DIM), jnp.float32),
        pltpu.VMEM((CHUNK, EMBED_DIM), jnp.float32),
        pltpu.SemaphoreType.DMA,
    ],
)
def _emb(x_hbm, table_hbm, pos_hbm, out_hbm, idx_v, pos_v, rows_v, sem):
    wid = lax.axis_index("s") * NC + lax.axis_index("c")
    pltpu.sync_copy(x_hbm.at[pl.ds(wid * CHUNKS_PER_W, CHUNKS_PER_W)], idx_v)
    pltpu.sync_copy(pos_hbm, pos_v)
    out_base = wid * ROWS_PER_W

    def chunk_body(c, _):
        pltpu.async_copy(table_hbm.at[idx_v.at[c]], rows_v, sem).wait()
        o = lax.rem(c * CHUNK, MAXLEN)

        def add_row(r, _):
            m = o + r
            rows_v[r, pl.ds(0, 16)] = rows_v[r, pl.ds(0, 16)] + pos_v[m, pl.ds(0, 16)]
            rows_v[r, pl.ds(16, 16)] = rows_v[r, pl.ds(16, 16)] + pos_v[m, pl.ds(16, 16)]
            return 0

        lax.fori_loop(0, CHUNK, add_row, 0)
        pltpu.sync_copy(rows_v, out_hbm.at[pl.ds(out_base + c * CHUNK, CHUNK)])
        return 0

    lax.fori_loop(0, CHUNKS_PER_W, chunk_body, 0)


def kernel(x, token_table, pos_table):
    x_flat = x.reshape(-1).astype(jnp.int32)
    x2d = x_flat.reshape(NW * CHUNKS_PER_W, CHUNK)
    pos_ext = jnp.concatenate([pos_table, pos_table[: POS_PAD - MAXLEN]], axis=0)
    out = _emb(x2d, token_table, pos_ext)
    return out.reshape(BATCH, MAXLEN, EMBED_DIM)


# speedup vs baseline: 1.2497x; 1.1839x over previous
import functools

import jax
import jax.numpy as jnp
from jax import lax
from jax.experimental import pallas as pl
from jax.experimental.pallas import tpu as pltpu
from jax.experimental.pallas import tpu_sc as plsc

MAXLEN = 200
EMBED_DIM = 32
BATCH = 4096

NC, NS = 2, 16
NW = NC * NS
TOTAL_ROWS = BATCH * MAXLEN
ROWS_PER_W = TOTAL_ROWS // NW
CHUNK = 128
CHUNKS_PER_W = ROWS_PER_W // CHUNK
POS_PAD = MAXLEN + CHUNK
NBUF = 4


@functools.partial(
    pl.kernel,
    out_type=jax.ShapeDtypeStruct((TOTAL_ROWS, EMBED_DIM), jnp.float32),
    mesh=plsc.VectorSubcoreMesh(core_axis_name="c", subcore_axis_name="s"),
    compiler_params=pltpu.CompilerParams(use_tc_tiling_on_sc=False),
    scratch_types=[
        pltpu.VMEM((CHUNKS_PER_W, CHUNK), jnp.int32),
        pltpu.VMEM((POS_PAD, EMBED_DIM), jnp.float32),
        pltpu.VMEM((NBUF, CHUNK, EMBED_DIM), jnp.float32),
        pltpu.VMEM((NBUF, CHUNK, EMBED_DIM), jnp.float32),
        pltpu.SemaphoreType.DMA((NBUF,)),
        pltpu.SemaphoreType.DMA((NBUF,)),
    ],
)
def _emb(x_hbm, table_hbm, pos_hbm, out_hbm, idx_v, pos_v, gbuf, wbuf, gsem, wsem):
    wid = lax.axis_index("s") * NC + lax.axis_index("c")
    pltpu.sync_copy(x_hbm.at[pl.ds(wid * CHUNKS_PER_W, CHUNKS_PER_W)], idx_v)
    pltpu.sync_copy(pos_hbm, pos_v)
    out_base = wid * ROWS_PER_W

    def start_gather(c, b):
        pltpu.make_async_copy(
            table_hbm.at[idx_v.at[c]], gbuf.at[b], gsem.at[b]
        ).start()

    for b in range(NBUF):
        start_gather(b, b)

    def group(g, _):
        for b in range(NBUF):
            c = g * NBUF + b
            pltpu.make_async_copy(
                table_hbm.at[idx_v.at[c]], gbuf.at[b], gsem.at[b]
            ).wait()

            @pl.when(g > 0)
            def _():
                pltpu.make_async_copy(
                    wbuf.at[b], out_hbm.at[pl.ds(out_base, CHUNK)], wsem.at[b]
                ).wait()

            o = lax.rem(c * CHUNK, MAXLEN)

            def add_row(r, _):
                m = o + r
                wbuf[b, r, pl.ds(0, 16)] = (
                    gbuf[b, r, pl.ds(0, 16)] + pos_v[m, pl.ds(0, 16)]
                )
                wbuf[b, r, pl.ds(16, 16)] = (
                    gbuf[b, r, pl.ds(16, 16)] + pos_v[m, pl.ds(16, 16)]
                )
                return 0

            lax.fori_loop(0, CHUNK, add_row, 0, unroll=4)

            pltpu.make_async_copy(
                wbuf.at[b], out_hbm.at[pl.ds(out_base + c * CHUNK, CHUNK)], wsem.at[b]
            ).start()

            @pl.when(c + NBUF < CHUNKS_PER_W)
            def _():
                start_gather(c + NBUF, b)

        return 0

    lax.fori_loop(0, CHUNKS_PER_W // NBUF, group, 0)

    for b in range(NBUF):
        pltpu.make_async_copy(
            wbuf.at[b], out_hbm.at[pl.ds(out_base, CHUNK)], wsem.at[b]
        ).wait()


def kernel(x, token_table, pos_table):
    x_flat = x.reshape(-1).astype(jnp.int32)
    x2d = x_flat.reshape(NW * CHUNKS_PER_W, CHUNK)
    pos_ext = jnp.concatenate([pos_table, pos_table[: POS_PAD - MAXLEN]], axis=0)
    out = _emb(x2d, token_table, pos_ext)
    return out.reshape(BATCH, MAXLEN, EMBED_DIM)
